# Initial kernel scaffold; baseline (speedup 1.0000x reference)
import jax
import jax.numpy as jnp
from jax.experimental import pallas as pl


def kernel(x, W_base, W_g, W_r, lora_A, lora_B):
    raise NotImplementedError("write your pallas kernel here")


# speedup vs baseline: 1.8332x; 1.8332x over previous
import functools

import jax
import jax.numpy as jnp
from jax.experimental import pallas as pl
from jax.experimental.pallas import tpu as pltpu

K_TOP = 2


def _router_kernel(x_ref, wg_ref, wr_ref, w_out_ref, acc_ref, *, nlt, inv_l):
    lt = pl.program_id(0)

    @pl.when(lt == 0)
    def _():
        acc_ref[...] = jnp.zeros_like(acc_ref)

    acc_ref[...] += jnp.sum(x_ref[...], axis=1)

    @pl.when(lt == nlt - 1)
    def _():
        pooled = acc_ref[...] * inv_l
        gated = jax.lax.dot_general(
            pooled, wg_ref[...], (((1,), (1,)), ((), ())),
            preferred_element_type=jnp.float32)
        logits = jax.lax.dot_general(
            gated, wr_ref[...], (((1,), (1,)), ((), ())),
            preferred_element_type=jnp.float32)
        m = jnp.max(logits, axis=-1, keepdims=True)
        p = jnp.exp(logits - m)
        probs = p / jnp.sum(p, axis=-1, keepdims=True)
        e_ids = jax.lax.broadcasted_iota(jnp.int32, probs.shape, 1)
        v1 = jnp.max(probs, axis=-1, keepdims=True)
        i1 = jnp.argmax(probs, axis=-1)[:, None]
        masked = jnp.where(e_ids == i1, -jnp.inf, probs)
        v2 = jnp.max(masked, axis=-1, keepdims=True)
        i2 = jnp.argmax(masked, axis=-1)[:, None]
        denom = v1 + v2
        w = jnp.where(e_ids == i1, v1 / denom, 0.0)
        w = jnp.where(e_ids == i2, v2 / denom, w)
        w_out_ref[...] = w.astype(w_out_ref.dtype)


def _combine_kernel(w_ref, a_ref, b_ref, wbt_ref, x_ref, out_ref, mt_ref, *,
                    r):
    b = pl.program_id(0)
    lt = pl.program_id(1)

    @pl.when(lt == 0)
    def _():
        w = w_ref[b, :]
        e = w.shape[0]
        k_exp = jax.lax.broadcasted_iota(jnp.int32, (e * r, e), 0) // r
        e_ids = jax.lax.broadcasted_iota(jnp.int32, (e * r, e), 1)
        sel = (k_exp == e_ids).astype(jnp.float32)
        w_rep = jnp.sum(sel * w[None, :], axis=1, keepdims=True)
        a_w = a_ref[...] * w_rep
        delta = jax.lax.dot_general(
            a_w, b_ref[...], (((0,), (0,)), ((), ())),
            preferred_element_type=jnp.float32)
        mt_ref[...] = wbt_ref[...] + delta

    xt = x_ref[0]
    out_ref[0] = jnp.dot(xt, mt_ref[...],
                         preferred_element_type=jnp.float32)


@jax.jit
def kernel(x, W_base, W_g, W_r, lora_A, lora_B):
    B, L, D = x.shape
    E, R, _ = lora_A.shape

    A_cat = lora_A.reshape(E * R, D)
    B_cat = lora_B.transpose(0, 2, 1).reshape(E * R, D)
    Wb_t = W_base.T

    TL_R = 2048
    nlt_r = L // TL_R
    weights = pl.pallas_call(
        functools.partial(_router_kernel, nlt=nlt_r, inv_l=1.0 / L),
        grid=(nlt_r,),
        in_specs=[
            pl.BlockSpec((B, TL_R, D), lambda lt: (0, lt, 0)),
            pl.BlockSpec((D, D), lambda lt: (0, 0)),
            pl.BlockSpec((E, D), lambda lt: (0, 0)),
        ],
        out_specs=pl.BlockSpec((B, E), lambda lt: (0, 0)),
        out_shape=jax.ShapeDtypeStruct((B, E), jnp.float32),
        scratch_shapes=[pltpu.VMEM((B, D), jnp.float32)],
    )(x, W_g, W_r)

    TL = 2048
    nlt = L // TL
    out = pl.pallas_call(
        functools.partial(_combine_kernel, r=R),
        grid=(B, nlt),
        in_specs=[
            pl.BlockSpec((B, E), lambda b, lt: (0, 0)),
            pl.BlockSpec((E * R, D), lambda b, lt: (0, 0)),
            pl.BlockSpec((E * R, D), lambda b, lt: (0, 0)),
            pl.BlockSpec((D, D), lambda b, lt: (0, 0)),
            pl.BlockSpec((1, TL, D), lambda b, lt: (b, lt, 0)),
        ],
        out_specs=pl.BlockSpec((1, TL, D), lambda b, lt: (b, lt, 0)),
        out_shape=jax.ShapeDtypeStruct((B, L, D), jnp.float32),
        scratch_shapes=[pltpu.VMEM((D, D), jnp.float32)],
    )(weights, A_cat, B_cat, Wb_t, x)

    return out
